# baseline (speedup 1.0000x reference)
import functools

import jax
import jax.numpy as jnp
from jax import lax
from jax.experimental import pallas as pl
from jax.experimental.pallas import tpu as pltpu
from jax.experimental.pallas import tpu_sc as plsc

_E = 320000
_D = 128
_NC = 2
_NS = 16
_NW = _NC * _NS
_PER = _E // _NW
_C = 400
_NCH = _PER // _C
_L = 16
_T = 64


def _prelude_body(wpad_ref, a0_ref, a1_ref, a2_ref, wcat_ref, idx_ref):
    row = lax.broadcasted_iota(jnp.int32, (_T, 1), 0)
    r0 = row // 12
    r1 = (row // 2) % 6
    r2 = row % 2
    acc = jnp.zeros((_T, _D), jnp.float32)
    for j in range(16):
        sel = ((j == r0).astype(jnp.float32)
               + (j == 5 + r1).astype(jnp.float32)
               + (j == 11 + r2).astype(jnp.float32))
        acc = acc + sel * wpad_ref[j, :][None, :]
    wcat_ref[...] = acc
    idx_ref[...] = (a0_ref[...] * 12 + a1_ref[...] * 2 + a2_ref[...]) * _D


_prelude = pl.pallas_call(
    _prelude_body,
    out_shape=(
        jax.ShapeDtypeStruct((_T, _D), jnp.float32),
        jax.ShapeDtypeStruct((_E // _D, _D), jnp.int32),
    ),
)


@functools.cache
def _build_sc_expand():
    mesh = plsc.VectorSubcoreMesh(
        core_axis_name="c", subcore_axis_name="s",
        num_cores=_NC, num_subcores=_NS)

    @functools.partial(
        pl.kernel,
        out_type=jax.ShapeDtypeStruct((_E * _D,), jnp.float32),
        mesh=mesh,
        compiler_params=pltpu.CompilerParams(needs_layout_passes=False),
        scratch_types=[
            pltpu.VMEM((_T * _D,), jnp.float32),
            pltpu.VMEM((_C,), jnp.int32),
            pltpu.VMEM((_C,), jnp.int32),
            pltpu.VMEM((_C * _D,), jnp.float32),
            pltpu.VMEM((_C * _D,), jnp.float32),
            pltpu.SemaphoreType.DMA,
            pltpu.SemaphoreType.DMA,
            pltpu.SemaphoreType.DMA,
            pltpu.SemaphoreType.DMA,
        ],
    )
    def _sc_expand(idx_hbm, wcat_hbm, out_hbm,
                   wcat_v, idx0_v, idx1_v, rows0_v, rows1_v,
                   asem0, asem1, osem0, osem1):
        wid = lax.axis_index("s") * _NC + lax.axis_index("c")
        base = wid * _PER

        bufs = ((idx0_v, rows0_v, asem0, osem0),
                (idx1_v, rows1_v, asem1, osem1))

        pltpu.async_copy(idx_hbm.at[pl.ds(base, _C)], idx0_v, asem0)
        pltpu.sync_copy(wcat_hbm, wcat_v)

        def run_chunk(i, b):
            idx_v, rows_v, asem, osem = bufs[b]
            oidx_v, _, oasem, _ = bufs[1 - b]
            eoff = base + i * _C

            pltpu.make_async_copy(
                idx_hbm.at[pl.ds(eoff, _C)], idx_v, asem).wait()

            @pl.when(i + 1 < _NCH)
            def _():
                pltpu.async_copy(
                    idx_hbm.at[pl.ds(eoff + _C, _C)], oidx_v, oasem)

            @pl.when(i >= 2)
            def _():
                pltpu.make_async_copy(
                    rows_v, out_hbm.at[pl.ds(eoff * _D, _C * _D)],
                    osem).wait()

            lanestep = lax.iota(jnp.int32, _L) * _D

            @plsc.parallel_loop(0, _C // _L, unroll=2)
            def _(j):
                cvec = idx_v[pl.ds(j * _L, _L)]
                dstb = lanestep + j * _L * _D
                for col in range(_D):
                    v = plsc.load_gather(wcat_v, [cvec + col])
                    plsc.store_scatter(rows_v, [dstb + col], v)

            pltpu.async_copy(
                rows_v, out_hbm.at[pl.ds(eoff * _D, _C * _D)], osem)

        def chunk(i, carry):
            @pl.when(lax.rem(i, 2) == 0)
            def _():
                run_chunk(i, 0)

            @pl.when(lax.rem(i, 2) == 1)
            def _():
                run_chunk(i, 1)

            return carry

        lax.fori_loop(0, _NCH, chunk, 0)

        pltpu.make_async_copy(
            rows0_v, out_hbm.at[pl.ds(base * _D, _C * _D)], osem0).wait()
        pltpu.make_async_copy(
            rows1_v, out_hbm.at[pl.ds(base * _D, _C * _D)], osem1).wait()

    return _sc_expand


def kernel(edge_attr, W0, W1, W2):
    ea = edge_attr.astype(jnp.int32)
    blk = (_E // _D, _D)
    wpad = jnp.concatenate(
        [W0, W1, W2, jnp.zeros((3, _D), jnp.float32)], axis=0)
    wcat, idx = _prelude(wpad,
                         ea[:, 0].reshape(blk),
                         ea[:, 1].reshape(blk),
                         ea[:, 2].reshape(blk))
    out_flat = _build_sc_expand()(idx.reshape(_E), wcat.reshape(_T * _D))
    return out_flat.reshape(_E, _D)


# speedup vs baseline: 4.7421x; 4.7421x over previous
import functools

import jax
import jax.numpy as jnp
from jax import lax
from jax.experimental import pallas as pl
from jax.experimental.pallas import tpu as pltpu
from jax.experimental.pallas import tpu_sc as plsc

_E = 320000
_D = 128
_NC = 2
_NS = 16
_NW = _NC * _NS
_PER = _E // _NW
_C = 400
_NCH = _PER // _C
_L = 16
_T = 64


def _prelude_body(wpad_ref, a0_ref, a1_ref, a2_ref, wcat_ref, idx_ref):
    row = lax.broadcasted_iota(jnp.int32, (_T, 1), 0)
    r0 = row // 12
    r1 = (row // 2) % 6
    r2 = row % 2
    acc = jnp.zeros((_T, _D), jnp.float32)
    for j in range(16):
        sel = ((j == r0).astype(jnp.float32)
               + (j == 5 + r1).astype(jnp.float32)
               + (j == 11 + r2).astype(jnp.float32))
        acc = acc + sel * wpad_ref[j, :][None, :]
    wcat_ref[...] = acc
    idx_ref[...] = (a0_ref[...] * 12 + a1_ref[...] * 2 + a2_ref[...]) * _D


_prelude = pl.pallas_call(
    _prelude_body,
    out_shape=(
        jax.ShapeDtypeStruct((_T, _D), jnp.float32),
        jax.ShapeDtypeStruct((_E // _D, _D), jnp.int32),
    ),
)


@functools.cache
def _build_sc_expand():
    mesh = plsc.VectorSubcoreMesh(
        core_axis_name="c", subcore_axis_name="s",
        num_cores=_NC, num_subcores=_NS)

    @functools.partial(
        pl.kernel,
        out_type=jax.ShapeDtypeStruct((_E * _D,), jnp.float32),
        mesh=mesh,
        scratch_types=[
            pltpu.VMEM((_T * _D,), jnp.float32),
            pltpu.VMEM((_C,), jnp.int32),
            pltpu.VMEM((_C,), jnp.int32),
            pltpu.VMEM((_C * _D,), jnp.float32),
            pltpu.VMEM((_C * _D,), jnp.float32),
            pltpu.SemaphoreType.DMA,
            pltpu.SemaphoreType.DMA,
            pltpu.SemaphoreType.DMA,
            pltpu.SemaphoreType.DMA,
        ],
    )
    def _sc_expand(idx_hbm, wcat_hbm, out_hbm,
                   wcat_v, idx0_v, idx1_v, rows0_v, rows1_v,
                   asem0, asem1, osem0, osem1):
        wid = lax.axis_index("s") * _NC + lax.axis_index("c")
        base = wid * _PER

        bufs = ((idx0_v, rows0_v, asem0, osem0),
                (idx1_v, rows1_v, asem1, osem1))

        pltpu.async_copy(idx_hbm.at[pl.ds(base, _C)], idx0_v, asem0)
        pltpu.sync_copy(wcat_hbm, wcat_v)

        def run_chunk(i, b):
            idx_v, rows_v, asem, osem = bufs[b]
            oidx_v, _, oasem, _ = bufs[1 - b]
            eoff = base + i * _C

            pltpu.make_async_copy(
                idx_hbm.at[pl.ds(eoff, _C)], idx_v, asem).wait()

            @pl.when(i + 1 < _NCH)
            def _():
                pltpu.async_copy(
                    idx_hbm.at[pl.ds(eoff + _C, _C)], oidx_v, oasem)

            @pl.when(i >= 2)
            def _():
                pltpu.make_async_copy(
                    rows_v, out_hbm.at[pl.ds(eoff * _D, _C * _D)],
                    osem).wait()

            @plsc.parallel_loop(0, _C // _L, unroll=3)
            def _(j):
                cvec = idx_v[pl.ds(j * _L, _L)]
                gb = j * _L * _D
                for m in range(_L):
                    c = cvec[m]
                    eb = gb + m * _D
                    for k in range(_D // _L):
                        rows_v[pl.ds(eb + k * _L, _L)] = (
                            wcat_v[pl.ds(c + k * _L, _L)])

            pltpu.async_copy(
                rows_v, out_hbm.at[pl.ds(eoff * _D, _C * _D)], osem)

        def chunk(i, carry):
            @pl.when(lax.rem(i, 2) == 0)
            def _():
                run_chunk(i, 0)

            @pl.when(lax.rem(i, 2) == 1)
            def _():
                run_chunk(i, 1)

            return carry

        lax.fori_loop(0, _NCH, chunk, 0)

        pltpu.make_async_copy(
            rows0_v, out_hbm.at[pl.ds(base * _D, _C * _D)], osem0).wait()
        pltpu.make_async_copy(
            rows1_v, out_hbm.at[pl.ds(base * _D, _C * _D)], osem1).wait()

    return _sc_expand


def kernel(edge_attr, W0, W1, W2):
    ea = edge_attr.astype(jnp.int32)
    blk = (_E // _D, _D)
    wpad = jnp.concatenate(
        [W0, W1, W2, jnp.zeros((3, _D), jnp.float32)], axis=0)
    wcat, idx = _prelude(wpad,
                         ea[:, 0].reshape(blk),
                         ea[:, 1].reshape(blk),
                         ea[:, 2].reshape(blk))
    out_flat = _build_sc_expand()(idx.reshape(_E), wcat.reshape(_T * _D))
    return out_flat.reshape(_E, _D)


# speedup vs baseline: 5.2994x; 1.1175x over previous
import functools

import jax
import jax.numpy as jnp
from jax import lax
from jax.experimental import pallas as pl
from jax.experimental.pallas import tpu as pltpu
from jax.experimental.pallas import tpu_sc as plsc

_E = 320000
_D = 128
_NC = 2
_NS = 16
_NW = _NC * _NS
_PER = _E // _NW
_C = 400
_NCH = _PER // _C
_L = 16
_T = 64


def _prelude_body(wpad_ref, a0_ref, a1_ref, a2_ref, wcat_ref, idx_ref):
    row = lax.broadcasted_iota(jnp.int32, (_T, 1), 0)
    r0 = row // 12
    r1 = (row // 2) % 6
    r2 = row % 2
    acc = jnp.zeros((_T, _D), jnp.float32)
    for j in range(16):
        sel = ((j == r0).astype(jnp.float32)
               + (j == 5 + r1).astype(jnp.float32)
               + (j == 11 + r2).astype(jnp.float32))
        acc = acc + sel * wpad_ref[j, :][None, :]
    wcat_ref[...] = acc
    idx_ref[...] = (a0_ref[...] * 12 + a1_ref[...] * 2 + a2_ref[...]) * _D


_prelude = pl.pallas_call(
    _prelude_body,
    out_shape=(
        jax.ShapeDtypeStruct((_T, _D), jnp.float32),
        jax.ShapeDtypeStruct((_E // _D, _D), jnp.int32),
    ),
)


@functools.cache
def _build_sc_expand():
    mesh = plsc.VectorSubcoreMesh(
        core_axis_name="c", subcore_axis_name="s",
        num_cores=_NC, num_subcores=_NS)

    @functools.partial(
        pl.kernel,
        out_type=jax.ShapeDtypeStruct((_E * _D,), jnp.float32),
        mesh=mesh,
        scratch_types=[
            pltpu.VMEM((_T * _D,), jnp.float32),
            pltpu.VMEM((_C,), jnp.int32),
            pltpu.VMEM((_C,), jnp.int32),
            pltpu.VMEM((_C * _D,), jnp.float32),
            pltpu.VMEM((_C * _D,), jnp.float32),
            pltpu.SemaphoreType.DMA,
            pltpu.SemaphoreType.DMA,
            pltpu.SemaphoreType.DMA,
            pltpu.SemaphoreType.DMA,
        ],
    )
    def _sc_expand(idx_hbm, wcat_hbm, out_hbm,
                   wcat_v, idx0_v, idx1_v, rows0_v, rows1_v,
                   asem0, asem1, osem0, osem1):
        wid = lax.axis_index("s") * _NC + lax.axis_index("c")
        base = wid * _PER

        bufs = ((idx0_v, rows0_v, asem0, osem0),
                (idx1_v, rows1_v, asem1, osem1))

        pltpu.async_copy(idx_hbm.at[pl.ds(base, _C)], idx0_v, asem0)
        pltpu.sync_copy(wcat_hbm, wcat_v)

        def run_chunk(i, b):
            idx_v, rows_v, asem, osem = bufs[b]
            oidx_v, _, oasem, _ = bufs[1 - b]
            eoff = base + i * _C

            pltpu.make_async_copy(
                idx_hbm.at[pl.ds(eoff, _C)], idx_v, asem).wait()

            @pl.when(i + 1 < _NCH)
            def _():
                pltpu.async_copy(
                    idx_hbm.at[pl.ds(eoff + _C, _C)], oidx_v, oasem)

            @pl.when(i >= 2)
            def _():
                pltpu.make_async_copy(
                    rows_v, out_hbm.at[pl.ds(eoff * _D, _C * _D)],
                    osem).wait()

            @plsc.parallel_loop(0, _C // _L, unroll=2)
            def _(j):
                cvec = idx_v[pl.ds(j * _L, _L)]
                gb = j * _L * _D
                cs = [cvec[m] for m in range(_L)]
                for k in range(_D // _L):
                    for m in range(_L):
                        rows_v[pl.ds(gb + m * _D + k * _L, _L)] = (
                            wcat_v[pl.ds(cs[m] + k * _L, _L)])

            pltpu.async_copy(
                rows_v, out_hbm.at[pl.ds(eoff * _D, _C * _D)], osem)

        def chunk(i, carry):
            @pl.when(lax.rem(i, 2) == 0)
            def _():
                run_chunk(i, 0)

            @pl.when(lax.rem(i, 2) == 1)
            def _():
                run_chunk(i, 1)

            return carry

        lax.fori_loop(0, _NCH, chunk, 0)

        pltpu.make_async_copy(
            rows0_v, out_hbm.at[pl.ds(base * _D, _C * _D)], osem0).wait()
        pltpu.make_async_copy(
            rows1_v, out_hbm.at[pl.ds(base * _D, _C * _D)], osem1).wait()

    return _sc_expand


def kernel(edge_attr, W0, W1, W2):
    ea = edge_attr.astype(jnp.int32)
    blk = (_E // _D, _D)
    wpad = jnp.concatenate(
        [W0, W1, W2, jnp.zeros((3, _D), jnp.float32)], axis=0)
    wcat, idx = _prelude(wpad,
                         ea[:, 0].reshape(blk),
                         ea[:, 1].reshape(blk),
                         ea[:, 2].reshape(blk))
    out_flat = _build_sc_expand()(idx.reshape(_E), wcat.reshape(_T * _D))
    return out_flat.reshape(_E, _D)


# speedup vs baseline: 8.7847x; 1.6577x over previous
import functools

import jax
import jax.numpy as jnp
from jax import lax
from jax.experimental import pallas as pl
from jax.experimental.pallas import tpu as pltpu
from jax.experimental.pallas import tpu_sc as plsc

_E = 320000
_D = 128
_NC = 2
_NS = 16
_NW = _NC * _NS
_PER = _E // _NW
_C = 400
_NCH = _PER // _C
_L = 16
_T = 64


def _prelude_body(wpad_ref, a0_ref, a1_ref, a2_ref, wcat_ref, idx_ref):
    row = lax.broadcasted_iota(jnp.int32, (_T, 1), 0)
    r0 = row // 12
    r1 = (row // 2) % 6
    r2 = row % 2
    acc = jnp.zeros((_T, _D), jnp.float32)
    for j in range(16):
        sel = ((j == r0).astype(jnp.float32)
               + (j == 5 + r1).astype(jnp.float32)
               + (j == 11 + r2).astype(jnp.float32))
        acc = acc + sel * wpad_ref[j, :][None, :]
    wcat_ref[...] = acc
    idx_ref[...] = (a0_ref[...] * 12 + a1_ref[...] * 2 + a2_ref[...]) * _D


_prelude = pl.pallas_call(
    _prelude_body,
    out_shape=(
        jax.ShapeDtypeStruct((_T, _D), jnp.float32),
        jax.ShapeDtypeStruct((_E // _D, _D), jnp.int32),
    ),
)


@functools.cache
def _build_sc_expand():
    mesh = plsc.VectorSubcoreMesh(
        core_axis_name="c", subcore_axis_name="s",
        num_cores=_NC, num_subcores=_NS)

    @functools.partial(
        pl.kernel,
        out_type=jax.ShapeDtypeStruct((_E * _D,), jnp.float32),
        mesh=mesh,
        scratch_types=[
            pltpu.VMEM((_T * _D,), jnp.float32),
            pltpu.VMEM((_C,), jnp.int32),
            pltpu.VMEM((_C,), jnp.int32),
            pltpu.VMEM((_C * _D,), jnp.float32),
            pltpu.VMEM((_C * _D,), jnp.float32),
            pltpu.SemaphoreType.DMA,
            pltpu.SemaphoreType.DMA,
            pltpu.SemaphoreType.DMA,
            pltpu.SemaphoreType.DMA,
        ],
    )
    def _sc_expand(idx_hbm, wcat_hbm, out_hbm,
                   wcat_v, idx0_v, idx1_v, rows0_v, rows1_v,
                   asem0, asem1, osem0, osem1):
        wid = lax.axis_index("s") * _NC + lax.axis_index("c")
        base = wid * _PER

        bufs = ((idx0_v, rows0_v, asem0, osem0),
                (idx1_v, rows1_v, asem1, osem1))

        pltpu.async_copy(idx_hbm.at[pl.ds(base, _C)], idx0_v, asem0)
        pltpu.sync_copy(wcat_hbm, wcat_v)

        def run_chunk(i, b):
            idx_v, rows_v, asem, osem = bufs[b]
            oidx_v, _, oasem, _ = bufs[1 - b]
            eoff = base + i * _C

            pltpu.make_async_copy(
                idx_hbm.at[pl.ds(eoff, _C)], idx_v, asem).wait()

            @pl.when(i + 1 < _NCH)
            def _():
                pltpu.async_copy(
                    idx_hbm.at[pl.ds(eoff + _C, _C)], oidx_v, oasem)

            @pl.when(i >= 2)
            def _():
                pltpu.make_async_copy(
                    rows_v, out_hbm.at[pl.ds(eoff * _D, _C * _D)],
                    osem).wait()

            @plsc.parallel_loop(0, _C // _L, unroll=2)
            def _(j):
                cvec = idx_v[pl.ds(j * _L, _L)]
                gb = j * _L * _D
                for m in range(_L):
                    c = cvec[m]
                    eb = gb + m * _D
                    vs = [wcat_v[pl.ds(c + k * _L, _L)]
                          for k in range(_D // _L)]
                    for k in range(_D // _L):
                        rows_v[pl.ds(eb + k * _L, _L)] = vs[k]

            pltpu.async_copy(
                rows_v, out_hbm.at[pl.ds(eoff * _D, _C * _D)], osem)

        def chunk(i, carry):
            @pl.when(lax.rem(i, 2) == 0)
            def _():
                run_chunk(i, 0)

            @pl.when(lax.rem(i, 2) == 1)
            def _():
                run_chunk(i, 1)

            return carry

        lax.fori_loop(0, _NCH, chunk, 0)

        pltpu.make_async_copy(
            rows0_v, out_hbm.at[pl.ds(base * _D, _C * _D)], osem0).wait()
        pltpu.make_async_copy(
            rows1_v, out_hbm.at[pl.ds(base * _D, _C * _D)], osem1).wait()

    return _sc_expand


def kernel(edge_attr, W0, W1, W2):
    ea = edge_attr.astype(jnp.int32)
    blk = (_E // _D, _D)
    wpad = jnp.concatenate(
        [W0, W1, W2, jnp.zeros((3, _D), jnp.float32)], axis=0)
    wcat, idx = _prelude(wpad,
                         ea[:, 0].reshape(blk),
                         ea[:, 1].reshape(blk),
                         ea[:, 2].reshape(blk))
    out_flat = _build_sc_expand()(idx.reshape(_E), wcat.reshape(_T * _D))
    return out_flat.reshape(_E, _D)


# speedup vs baseline: 9.4646x; 1.0774x over previous
import functools

import jax
import jax.numpy as jnp
from jax import lax
from jax.experimental import pallas as pl
from jax.experimental.pallas import tpu as pltpu
from jax.experimental.pallas import tpu_sc as plsc

_E = 320000
_D = 128
_NC = 2
_NS = 16
_NW = _NC * _NS
_PER = _E // _NW
_C = 400
_NCH = _PER // _C
_L = 16
_T = 64


def _prelude_body(wpad_ref, a0_ref, a1_ref, a2_ref, wcat_ref, idx_ref):
    row = lax.broadcasted_iota(jnp.int32, (_T, 1), 0)
    r0 = row // 12
    r1 = (row // 2) % 6
    r2 = row % 2
    acc = jnp.zeros((_T, _D), jnp.float32)
    for j in range(16):
        sel = ((j == r0).astype(jnp.float32)
               + (j == 5 + r1).astype(jnp.float32)
               + (j == 11 + r2).astype(jnp.float32))
        acc = acc + sel * wpad_ref[j, :][None, :]
    wcat_ref[...] = acc
    idx_ref[...] = (a0_ref[...] * 12 + a1_ref[...] * 2 + a2_ref[...]) * _D


_prelude = pl.pallas_call(
    _prelude_body,
    out_shape=(
        jax.ShapeDtypeStruct((_T, _D), jnp.float32),
        jax.ShapeDtypeStruct((_E // _D, _D), jnp.int32),
    ),
)


@functools.cache
def _build_sc_expand():
    mesh = plsc.VectorSubcoreMesh(
        core_axis_name="c", subcore_axis_name="s",
        num_cores=_NC, num_subcores=_NS)

    @functools.partial(
        pl.kernel,
        out_type=jax.ShapeDtypeStruct((_E * _D,), jnp.float32),
        mesh=mesh,
        scratch_types=[
            pltpu.VMEM((_T * _D,), jnp.float32),
            pltpu.VMEM((_C,), jnp.int32),
            pltpu.VMEM((_C,), jnp.int32),
            pltpu.VMEM((_C * _D,), jnp.float32),
            pltpu.VMEM((_C * _D,), jnp.float32),
            pltpu.SemaphoreType.DMA,
            pltpu.SemaphoreType.DMA,
            pltpu.SemaphoreType.DMA,
            pltpu.SemaphoreType.DMA,
        ],
    )
    def _sc_expand(idx_hbm, wcat_hbm, out_hbm,
                   wcat_v, idx0_v, idx1_v, rows0_v, rows1_v,
                   asem0, asem1, osem0, osem1):
        wid = lax.axis_index("s") * _NC + lax.axis_index("c")
        base = wid * _PER

        bufs = ((idx0_v, rows0_v, asem0, osem0),
                (idx1_v, rows1_v, asem1, osem1))

        pltpu.async_copy(idx_hbm.at[pl.ds(base, _C)], idx0_v, asem0)
        pltpu.sync_copy(wcat_hbm, wcat_v)

        def run_chunk(i, b):
            idx_v, rows_v, asem, osem = bufs[b]
            oidx_v, _, oasem, _ = bufs[1 - b]
            eoff = base + i * _C

            pltpu.make_async_copy(
                idx_hbm.at[pl.ds(eoff, _C)], idx_v, asem).wait()

            @pl.when(i + 1 < _NCH)
            def _():
                pltpu.async_copy(
                    idx_hbm.at[pl.ds(eoff + _C, _C)], oidx_v, oasem)

            @pl.when(i >= 2)
            def _():
                pltpu.make_async_copy(
                    rows_v, out_hbm.at[pl.ds(eoff * _D, _C * _D)],
                    osem).wait()

            @plsc.parallel_loop(0, _C // _L, unroll=2)
            def _(j):
                cvec = idx_v[pl.ds(j * _L, _L)]
                gb = j * _L * _D
                for m in range(0, _L, 2):
                    c0 = cvec[m]
                    c1 = cvec[m + 1]
                    eb0 = gb + m * _D
                    eb1 = eb0 + _D
                    vs0 = [wcat_v[pl.ds(c0 + k * _L, _L)]
                           for k in range(_D // _L)]
                    vs1 = [wcat_v[pl.ds(c1 + k * _L, _L)]
                           for k in range(_D // _L)]
                    for k in range(_D // _L):
                        rows_v[pl.ds(eb0 + k * _L, _L)] = vs0[k]
                    for k in range(_D // _L):
                        rows_v[pl.ds(eb1 + k * _L, _L)] = vs1[k]

            pltpu.async_copy(
                rows_v, out_hbm.at[pl.ds(eoff * _D, _C * _D)], osem)

        def chunk(i, carry):
            @pl.when(lax.rem(i, 2) == 0)
            def _():
                run_chunk(i, 0)

            @pl.when(lax.rem(i, 2) == 1)
            def _():
                run_chunk(i, 1)

            return carry

        lax.fori_loop(0, _NCH, chunk, 0)

        pltpu.make_async_copy(
            rows0_v, out_hbm.at[pl.ds(base * _D, _C * _D)], osem0).wait()
        pltpu.make_async_copy(
            rows1_v, out_hbm.at[pl.ds(base * _D, _C * _D)], osem1).wait()

    return _sc_expand


def kernel(edge_attr, W0, W1, W2):
    ea = edge_attr.astype(jnp.int32)
    blk = (_E // _D, _D)
    wpad = jnp.concatenate(
        [W0, W1, W2, jnp.zeros((3, _D), jnp.float32)], axis=0)
    wcat, idx = _prelude(wpad,
                         ea[:, 0].reshape(blk),
                         ea[:, 1].reshape(blk),
                         ea[:, 2].reshape(blk))
    out_flat = _build_sc_expand()(idx.reshape(_E), wcat.reshape(_T * _D))
    return out_flat.reshape(_E, _D)


# speedup vs baseline: 9.5398x; 1.0079x over previous
import functools

import jax
import jax.numpy as jnp
from jax import lax
from jax.experimental import pallas as pl
from jax.experimental.pallas import tpu as pltpu
from jax.experimental.pallas import tpu_sc as plsc

_E = 320000
_D = 128
_NC = 2
_NS = 16
_NW = _NC * _NS
_PER = _E // _NW
_C = 400
_NCH = _PER // _C
_L = 16
_T = 64


def _prelude_body(wpad_ref, a0_ref, a1_ref, a2_ref, wcat_ref, idx_ref):
    row = lax.broadcasted_iota(jnp.int32, (_T, 1), 0)
    r0 = row // 12
    r1 = (row // 2) % 6
    r2 = row % 2
    acc = jnp.zeros((_T, _D), jnp.float32)
    for j in range(16):
        sel = ((j == r0).astype(jnp.float32)
               + (j == 5 + r1).astype(jnp.float32)
               + (j == 11 + r2).astype(jnp.float32))
        acc = acc + sel * wpad_ref[j, :][None, :]
    wcat_ref[...] = acc
    idx_ref[...] = (a0_ref[...] * 12 + a1_ref[...] * 2 + a2_ref[...]) * _D


_prelude = pl.pallas_call(
    _prelude_body,
    out_shape=(
        jax.ShapeDtypeStruct((_T, _D), jnp.float32),
        jax.ShapeDtypeStruct((_E // _D, _D), jnp.int32),
    ),
)


@functools.cache
def _build_sc_expand():
    mesh = plsc.VectorSubcoreMesh(
        core_axis_name="c", subcore_axis_name="s",
        num_cores=_NC, num_subcores=_NS)

    @functools.partial(
        pl.kernel,
        out_type=jax.ShapeDtypeStruct((_E * _D,), jnp.float32),
        mesh=mesh,
        scratch_types=[
            pltpu.VMEM((_T * _D,), jnp.float32),
            pltpu.VMEM((_C,), jnp.int32),
            pltpu.VMEM((_C,), jnp.int32),
            pltpu.VMEM((_C * _D,), jnp.float32),
            pltpu.VMEM((_C * _D,), jnp.float32),
            pltpu.SemaphoreType.DMA,
            pltpu.SemaphoreType.DMA,
            pltpu.SemaphoreType.DMA,
            pltpu.SemaphoreType.DMA,
        ],
    )
    def _sc_expand(idx_hbm, wcat_hbm, out_hbm,
                   wcat_v, idx0_v, idx1_v, rows0_v, rows1_v,
                   asem0, asem1, osem0, osem1):
        wid = lax.axis_index("s") * _NC + lax.axis_index("c")
        base = wid * _PER

        bufs = ((idx0_v, rows0_v, asem0, osem0),
                (idx1_v, rows1_v, asem1, osem1))

        pltpu.async_copy(idx_hbm.at[pl.ds(base, _C)], idx0_v, asem0)
        pltpu.sync_copy(wcat_hbm, wcat_v)

        def run_chunk(i, b):
            idx_v, rows_v, asem, osem = bufs[b]
            oidx_v, _, oasem, _ = bufs[1 - b]
            eoff = base + i * _C

            pltpu.make_async_copy(
                idx_hbm.at[pl.ds(eoff, _C)], idx_v, asem).wait()

            @pl.when(i + 1 < _NCH)
            def _():
                pltpu.async_copy(
                    idx_hbm.at[pl.ds(eoff + _C, _C)], oidx_v, oasem)

            @pl.when(i >= 2)
            def _():
                pltpu.make_async_copy(
                    rows_v, out_hbm.at[pl.ds(eoff * _D, _C * _D)],
                    osem).wait()

            @plsc.parallel_loop(0, _C // _L, unroll=2)
            def _(j):
                cvec = idx_v[pl.ds(j * _L, _L)]
                gb = j * _L * _D
                for m in range(0, _L, 4):
                    cs = [cvec[m + t] for t in range(4)]
                    ebs = [gb + (m + t) * _D for t in range(4)]
                    vss = [[wcat_v[pl.ds(cs[t] + k * _L, _L)]
                            for k in range(_D // _L)] for t in range(4)]
                    for t in range(4):
                        for k in range(_D // _L):
                            rows_v[pl.ds(ebs[t] + k * _L, _L)] = vss[t][k]

            pltpu.async_copy(
                rows_v, out_hbm.at[pl.ds(eoff * _D, _C * _D)], osem)

        def chunk(i, carry):
            @pl.when(lax.rem(i, 2) == 0)
            def _():
                run_chunk(i, 0)

            @pl.when(lax.rem(i, 2) == 1)
            def _():
                run_chunk(i, 1)

            return carry

        lax.fori_loop(0, _NCH, chunk, 0)

        pltpu.make_async_copy(
            rows0_v, out_hbm.at[pl.ds(base * _D, _C * _D)], osem0).wait()
        pltpu.make_async_copy(
            rows1_v, out_hbm.at[pl.ds(base * _D, _C * _D)], osem1).wait()

    return _sc_expand


def kernel(edge_attr, W0, W1, W2):
    ea = edge_attr.astype(jnp.int32)
    blk = (_E // _D, _D)
    wpad = jnp.concatenate(
        [W0, W1, W2, jnp.zeros((3, _D), jnp.float32)], axis=0)
    wcat, idx = _prelude(wpad,
                         ea[:, 0].reshape(blk),
                         ea[:, 1].reshape(blk),
                         ea[:, 2].reshape(blk))
    out_flat = _build_sc_expand()(idx.reshape(_E), wcat.reshape(_T * _D))
    return out_flat.reshape(_E, _D)


# speedup vs baseline: 9.9149x; 1.0393x over previous
import functools

import jax
import jax.numpy as jnp
from jax import lax
from jax.experimental import pallas as pl
from jax.experimental.pallas import tpu as pltpu
from jax.experimental.pallas import tpu_sc as plsc

_E = 320000
_D = 128
_NC = 2
_NS = 16
_NW = _NC * _NS
_PER = _E // _NW
_C = 400
_NCH = _PER // _C
_L = 16
_T = 64


def _prelude_body(wpad_ref, a0_ref, a1_ref, a2_ref, wcat_ref, idx_ref):
    row = lax.broadcasted_iota(jnp.int32, (_T, 1), 0)
    r0 = row // 12
    r1 = (row // 2) % 6
    r2 = row % 2
    acc = jnp.zeros((_T, _D), jnp.float32)
    for j in range(16):
        sel = ((j == r0).astype(jnp.float32)
               + (j == 5 + r1).astype(jnp.float32)
               + (j == 11 + r2).astype(jnp.float32))
        acc = acc + sel * wpad_ref[j, :][None, :]
    wcat_ref[...] = acc
    idx_ref[...] = (a0_ref[...] * 12 + a1_ref[...] * 2 + a2_ref[...]) * _D


_prelude = pl.pallas_call(
    _prelude_body,
    out_shape=(
        jax.ShapeDtypeStruct((_T, _D), jnp.float32),
        jax.ShapeDtypeStruct((_E // _D, _D), jnp.int32),
    ),
)


@functools.cache
def _build_sc_expand():
    mesh = plsc.VectorSubcoreMesh(
        core_axis_name="c", subcore_axis_name="s",
        num_cores=_NC, num_subcores=_NS)

    @functools.partial(
        pl.kernel,
        out_type=jax.ShapeDtypeStruct((_E * _D,), jnp.float32),
        mesh=mesh,
        scratch_types=[
            pltpu.VMEM((_T * _D,), jnp.float32),
            pltpu.VMEM((_C,), jnp.int32),
            pltpu.VMEM((_C,), jnp.int32),
            pltpu.VMEM((_C * _D,), jnp.float32),
            pltpu.VMEM((_C * _D,), jnp.float32),
            pltpu.SemaphoreType.DMA,
            pltpu.SemaphoreType.DMA,
            pltpu.SemaphoreType.DMA,
            pltpu.SemaphoreType.DMA,
        ],
    )
    def _sc_expand(idx_hbm, wcat_hbm, out_hbm,
                   wcat_v, idx0_v, idx1_v, rows0_v, rows1_v,
                   asem0, asem1, osem0, osem1):
        wid = lax.axis_index("s") * _NC + lax.axis_index("c")
        base = wid * _PER

        bufs = ((idx0_v, rows0_v, asem0, osem0),
                (idx1_v, rows1_v, asem1, osem1))

        pltpu.async_copy(idx_hbm.at[pl.ds(base, _C)], idx0_v, asem0)
        pltpu.sync_copy(wcat_hbm, wcat_v)

        def run_chunk(i, b):
            idx_v, rows_v, asem, osem = bufs[b]
            oidx_v, _, oasem, _ = bufs[1 - b]
            eoff = base + i * _C

            pltpu.make_async_copy(
                idx_hbm.at[pl.ds(eoff, _C)], idx_v, asem).wait()

            @pl.when(i + 1 < _NCH)
            def _():
                pltpu.async_copy(
                    idx_hbm.at[pl.ds(eoff + _C, _C)], oidx_v, oasem)

            @pl.when(i >= 2)
            def _():
                pltpu.make_async_copy(
                    rows_v, out_hbm.at[pl.ds(eoff * _D, _C * _D)],
                    osem).wait()

            @plsc.parallel_loop(0, _C // _L, unroll=2)
            def _(j):
                cvec = idx_v[pl.ds(j * _L, _L)]
                gb = j * _L * _D
                call = [cvec[m] for m in range(_L)]
                for m in range(0, _L, 4):
                    ebs = [gb + (m + t) * _D for t in range(4)]
                    vss = [[wcat_v[pl.ds(call[m + t] + k * _L, _L)]
                            for k in range(_D // _L)] for t in range(4)]
                    for t in range(4):
                        for k in range(_D // _L):
                            rows_v[pl.ds(ebs[t] + k * _L, _L)] = vss[t][k]

            pltpu.async_copy(
                rows_v, out_hbm.at[pl.ds(eoff * _D, _C * _D)], osem)

        def chunk(i, carry):
            @pl.when(lax.rem(i, 2) == 0)
            def _():
                run_chunk(i, 0)

            @pl.when(lax.rem(i, 2) == 1)
            def _():
                run_chunk(i, 1)

            return carry

        lax.fori_loop(0, _NCH, chunk, 0)

        pltpu.make_async_copy(
            rows0_v, out_hbm.at[pl.ds(base * _D, _C * _D)], osem0).wait()
        pltpu.make_async_copy(
            rows1_v, out_hbm.at[pl.ds(base * _D, _C * _D)], osem1).wait()

    return _sc_expand


def kernel(edge_attr, W0, W1, W2):
    ea = edge_attr.astype(jnp.int32)
    blk = (_E // _D, _D)
    wpad = jnp.concatenate(
        [W0, W1, W2, jnp.zeros((3, _D), jnp.float32)], axis=0)
    wcat, idx = _prelude(wpad,
                         ea[:, 0].reshape(blk),
                         ea[:, 1].reshape(blk),
                         ea[:, 2].reshape(blk))
    out_flat = _build_sc_expand()(idx.reshape(_E), wcat.reshape(_T * _D))
    return out_flat.reshape(_E, _D)
